# baseline (speedup 1.0000x reference)
import functools

import jax
import jax.numpy as jnp
from jax import lax
from jax.experimental import pallas as pl
from jax.experimental.pallas import tpu as pltpu
from jax.experimental.pallas import tpu_sc as plsc

D = 2048; NH = 32; NKV = 8; HD = 64; E = 8; INTER = 1024; T = 256
EPS = 1e-06; THETA = 10000.0; SW = 127; LIMIT = 7.0; ALPHA = 1.702
GS = NH // NKV; HALF = HD // 2; SCALE = HD ** -0.5

BF = jnp.bfloat16
F32 = jnp.float32

QKV_OUT = NH * HD + 2 * NKV * HD
QKV_BLK = 512
OW_BLK = 512
F_BLK = 512
NF = INTER // F_BLK


def _rms_scale(x):
    return lax.rsqrt(jnp.mean(x * x, axis=-1, keepdims=True) + EPS)


def _qkv_body(x_ref, ln_ref, w_ref, b_ref, o_ref):
    x = x_ref[...]
    h = (x * _rms_scale(x) * ln_ref[...]).astype(BF)
    w = w_ref[...].astype(BF)
    acc = lax.dot_general(h, w, (((1,), (1,)), ((), ())),
                          preferred_element_type=F32)
    o_ref[...] = acc + b_ref[...]


def _qkv_call(hidden, ln1_w, qkv_w, qkv_b):
    grid = (QKV_OUT // QKV_BLK,)
    return pl.pallas_call(
        _qkv_body,
        grid=grid,
        in_specs=[
            pl.BlockSpec((T, D), lambda b: (0, 0)),
            pl.BlockSpec((1, D), lambda b: (0, 0)),
            pl.BlockSpec((QKV_BLK, D), lambda b: (b, 0)),
            pl.BlockSpec((1, QKV_BLK), lambda b: (0, b)),
        ],
        out_specs=pl.BlockSpec((T, QKV_BLK), lambda b: (0, b)),
        out_shape=jax.ShapeDtypeStruct((T, QKV_OUT), F32),
    )(hidden, ln1_w.reshape(1, D), qkv_w, qkv_b.reshape(1, QKV_OUT))


def _rope(x, cos, sin):
    x1 = x[:, :HALF]
    x2 = x[:, HALF:]
    return jnp.concatenate([x1 * cos - x2 * sin, x2 * cos + x1 * sin], axis=1)


_KVPG = 2


def _attn_body(q_ref, k_ref, v_ref, cos_ref, sin_ref, sinks_ref, o_ref):
    cos = cos_ref[...]
    sin = sin_ref[...]
    ii = lax.broadcasted_iota(jnp.int32, (T, T), 0)
    jj = lax.broadcasted_iota(jnp.int32, (T, T), 1)
    mask = (jj <= ii) & ((ii - jj) <= SW)
    for kk in range(_KVPG):
        k = _rope(k_ref[:, kk * HD:(kk + 1) * HD], cos, sin).astype(BF)
        v = v_ref[:, kk * HD:(kk + 1) * HD].astype(BF)
        for i in range(GS):
            h = kk * GS + i
            q = _rope(q_ref[:, h * HD:(h + 1) * HD], cos, sin).astype(BF)
            logits = lax.dot_general(q, k, (((1,), (1,)), ((), ())),
                                     preferred_element_type=F32) * SCALE
            logits = jnp.where(mask, logits, -1e30)
            s = sinks_ref[h, 0]
            m = jnp.maximum(jnp.max(logits, axis=1, keepdims=True), s)
            p = jnp.exp(logits - m)
            denom = jnp.sum(p, axis=1, keepdims=True) + jnp.exp(s - m)
            probs = (p / denom).astype(BF)
            o_ref[:, h * HD:(h + 1) * HD] = jnp.dot(
                probs, v, preferred_element_type=F32)


def _attn_call(qkv, cos, sin, sinks):
    grid = (NKV // _KVPG,)
    qw = _KVPG * GS * HD
    kvw = _KVPG * HD
    kv_col0 = (NH * HD) // kvw
    v_col0 = (NH * HD + NKV * HD) // kvw
    return pl.pallas_call(
        _attn_body,
        grid=grid,
        in_specs=[
            pl.BlockSpec((T, qw), lambda g: (0, g)),
            pl.BlockSpec((T, kvw), lambda g: (0, kv_col0 + g)),
            pl.BlockSpec((T, kvw), lambda g: (0, v_col0 + g)),
            pl.BlockSpec((T, HALF), lambda g: (0, 0)),
            pl.BlockSpec((T, HALF), lambda g: (0, 0)),
            pl.BlockSpec((_KVPG * GS, 1), lambda g: (g, 0)),
        ],
        out_specs=pl.BlockSpec((T, qw), lambda g: (0, g)),
        out_shape=jax.ShapeDtypeStruct((T, NH * HD), F32),
    )(qkv, qkv, qkv, cos, sin, sinks.reshape(NH, 1))


def _oproj_body(attn_ref, w_ref, b_ref, hid_ref, o_ref):
    a = attn_ref[...].astype(BF)
    w = w_ref[...].astype(BF)
    acc = lax.dot_general(a, w, (((1,), (1,)), ((), ())),
                          preferred_element_type=F32)
    o_ref[...] = hid_ref[...] + acc + b_ref[...]


def _oproj_call(attn, o_w, o_b, hidden):
    grid = (D // OW_BLK,)
    return pl.pallas_call(
        _oproj_body,
        grid=grid,
        in_specs=[
            pl.BlockSpec((T, NH * HD), lambda b: (0, 0)),
            pl.BlockSpec((OW_BLK, NH * HD), lambda b: (b, 0)),
            pl.BlockSpec((1, OW_BLK), lambda b: (0, b)),
            pl.BlockSpec((T, OW_BLK), lambda b: (0, b)),
        ],
        out_specs=pl.BlockSpec((T, OW_BLK), lambda b: (0, b)),
        out_shape=jax.ShapeDtypeStruct((T, D), F32),
    )(attn, o_w, o_b.reshape(1, D), hidden)


def _router_body(res_ref, ln_ref, rw_ref, rb_ref, h2_ref, rl_ref):
    x = res_ref[...]
    h2 = x * _rms_scale(x) * ln_ref[...]
    h2_ref[...] = h2
    rw = rw_ref[...].astype(BF)
    rl_ref[...] = lax.dot_general(h2.astype(BF), rw, (((1,), (1,)), ((), ())),
                                  preferred_element_type=F32) + rb_ref[...]


def _router_call(res1, ln2_w, router_w, router_b):
    return pl.pallas_call(
        _router_body,
        out_shape=[
            jax.ShapeDtypeStruct((T, D), F32),
            jax.ShapeDtypeStruct((T, E), F32),
        ],
    )(res1, ln2_w.reshape(1, D), router_w, router_b.reshape(1, E))


_NC = 2
_NTW = 16
_TKW = T // _NTW


def _route_body(rl_hbm, c_hbm, rl_v, c_v):
    wid = lax.axis_index("s") * _NC + lax.axis_index("c")

    @pl.when(wid < _NTW)
    def _work():
        base = wid * _TKW
        pltpu.sync_copy(rl_hbm, rl_v)
        neg = jnp.full((_TKW,), -1e30, F32)
        m1 = neg
        m2 = neg
        i1 = jnp.zeros((_TKW,), jnp.int32)
        i2 = jnp.full((_TKW,), -1, jnp.int32)
        for e in range(E):
            v = rl_v[pl.ds(e * T + base, _TKW)]
            cand_i = jnp.where(v > m1, i1, e)
            cand_v = jnp.minimum(v, m1)
            i1 = jnp.where(v > m1, e, i1)
            m1 = jnp.maximum(m1, v)
            i2 = jnp.where(cand_v > m2, cand_i, i2)
            m2 = jnp.maximum(m2, cand_v)
        ed = jnp.exp(m2 - m1)
        w1 = 1.0 / (1.0 + ed)
        w2 = 1.0 - w1
        zero = jnp.zeros((_TKW,), F32)
        for e in range(E):
            ce = jnp.where(i1 == e, w1, zero) + jnp.where(i2 == e, w2, zero)
            c_v[pl.ds(e * _TKW, _TKW)] = ce
        for e in range(E):
            pltpu.sync_copy(c_v.at[pl.ds(e * _TKW, _TKW)],
                            c_hbm.at[pl.ds(e * T + base, _TKW)])


def _route_call(rl_t_flat):
    mesh = plsc.VectorSubcoreMesh(core_axis_name="c", subcore_axis_name="s")
    fn = functools.partial(
        pl.kernel, mesh=mesh,
        out_type=jax.ShapeDtypeStruct((E * T,), F32),
        scratch_types=[
            pltpu.VMEM((E * T,), F32),
            pltpu.VMEM((E * _TKW,), F32),
        ],
    )(_route_body)
    return fn(rl_t_flat)


def _moe_body(h2_ref, res_ref, c_ref, wg_ref, wu_ref, bg_ref, bu_ref,
              wd_ref, bd_ref, o_ref):
    e = pl.program_id(0)
    f = pl.program_id(1)

    @pl.when((e == 0) & (f == 0))
    def _init():
        o_ref[...] = res_ref[...]

    h2 = h2_ref[...].astype(BF)
    gate = lax.dot_general(h2, wg_ref[0].astype(BF), (((1,), (0,)), ((), ())),
                           preferred_element_type=F32) + bg_ref[0]
    up = lax.dot_general(h2, wu_ref[0].astype(BF), (((1,), (0,)), ((), ())),
                         preferred_element_type=F32) + bu_ref[0]
    gate = jnp.minimum(gate, LIMIT)
    up = jnp.clip(up, -LIMIT, LIMIT)
    act = ((up + 1.0) * (gate * jax.nn.sigmoid(ALPHA * gate))).astype(BF)
    downp = lax.dot_general(act, wd_ref[0].astype(BF), (((1,), (0,)), ((), ())),
                            preferred_element_type=F32)

    eidx = lax.broadcasted_iota(jnp.int32, (T, E), 1)
    ccol = jnp.sum(jnp.where(eidx == e, c_ref[...], 0.0),
                   axis=1, keepdims=True)
    contrib = ccol * downp

    @pl.when(f == 0)
    def _bias():
        o_ref[...] += ccol * bd_ref[0]

    o_ref[...] += contrib


def _moe_call(h2, res1, c, w_gate_up, b_gate_up, w_down, b_down):
    grid = (E, NF)
    return pl.pallas_call(
        _moe_body,
        grid=grid,
        in_specs=[
            pl.BlockSpec((T, D), lambda e, f: (0, 0)),
            pl.BlockSpec((T, D), lambda e, f: (0, 0)),
            pl.BlockSpec((T, E), lambda e, f: (0, 0)),
            pl.BlockSpec((1, D, F_BLK), lambda e, f: (e, 0, f)),
            pl.BlockSpec((1, D, F_BLK), lambda e, f: (e, 0, NF + f)),
            pl.BlockSpec((1, 1, F_BLK), lambda e, f: (e, 0, f)),
            pl.BlockSpec((1, 1, F_BLK), lambda e, f: (e, 0, NF + f)),
            pl.BlockSpec((1, F_BLK, D), lambda e, f: (e, f, 0)),
            pl.BlockSpec((1, 1, D), lambda e, f: (e, 0, 0)),
        ],
        out_specs=pl.BlockSpec((T, D), lambda e, f: (0, 0)),
        out_shape=jax.ShapeDtypeStruct((T, D), F32),
    )(h2, res1, c, w_gate_up, w_gate_up,
      b_gate_up.reshape(E, 1, 2 * INTER), b_gate_up.reshape(E, 1, 2 * INTER),
      w_down, b_down.reshape(E, 1, D))


def kernel(positions, hidden_states, ln1_w, qkv_w, qkv_b, sinks, o_w, o_b,
           ln2_w, router_w, router_b, w_gate_up, b_gate_up, w_down, b_down):
    inv = 1.0 / (THETA ** (jnp.arange(HALF, dtype=F32) / HALF))
    ang = positions.astype(F32)[:, None] * inv[None, :]
    cos = jnp.cos(ang)
    sin = jnp.sin(ang)

    qkv = _qkv_call(hidden_states, ln1_w, qkv_w, qkv_b)
    attn = _attn_call(qkv, cos, sin, sinks)
    res1 = _oproj_call(attn, o_w, o_b, hidden_states)
    h2, rl = _router_call(res1, ln2_w, router_w, router_b)
    c = _route_call(rl.T.reshape(-1)).reshape(E, T).T
    return _moe_call(h2, res1, c, w_gate_up, b_gate_up, w_down, b_down)


# speedup vs baseline: 1.0526x; 1.0526x over previous
import functools

import jax
import jax.numpy as jnp
from jax import lax
from jax.experimental import pallas as pl
from jax.experimental.pallas import tpu as pltpu
from jax.experimental.pallas import tpu_sc as plsc

D = 2048; NH = 32; NKV = 8; HD = 64; E = 8; INTER = 1024; T = 256
EPS = 1e-06; THETA = 10000.0; SW = 127; LIMIT = 7.0; ALPHA = 1.702
GS = NH // NKV; HALF = HD // 2; SCALE = HD ** -0.5

BF = jnp.bfloat16
F32 = jnp.float32

QKV_OUT = NH * HD + 2 * NKV * HD
QKV_BLK = 512
OW_BLK = 512
F_BLK = 512
NF = INTER // F_BLK


def _rms_scale(x):
    return lax.rsqrt(jnp.mean(x * x, axis=-1, keepdims=True) + EPS)


def _qkv_body(x_ref, ln_ref, w_ref, b_ref, o_ref):
    x = x_ref[...]
    h = (x * _rms_scale(x) * ln_ref[...]).astype(BF)
    w = w_ref[...].astype(BF)
    acc = lax.dot_general(h, w, (((1,), (1,)), ((), ())),
                          preferred_element_type=F32)
    o_ref[...] = acc + b_ref[...]


def _qkv_call(hidden, ln1_w, qkv_w, qkv_b):
    grid = (QKV_OUT // QKV_BLK,)
    return pl.pallas_call(
        _qkv_body,
        grid=grid,
        in_specs=[
            pl.BlockSpec((T, D), lambda b: (0, 0)),
            pl.BlockSpec((1, D), lambda b: (0, 0)),
            pl.BlockSpec((QKV_BLK, D), lambda b: (b, 0)),
            pl.BlockSpec((1, QKV_BLK), lambda b: (0, b)),
        ],
        out_specs=pl.BlockSpec((T, QKV_BLK), lambda b: (0, b)),
        out_shape=jax.ShapeDtypeStruct((T, QKV_OUT), F32),
    )(hidden, ln1_w.reshape(1, D), qkv_w, qkv_b.reshape(1, QKV_OUT))


def _rope(x, cos, sin):
    x1 = x[:, :HALF]
    x2 = x[:, HALF:]
    return jnp.concatenate([x1 * cos - x2 * sin, x2 * cos + x1 * sin], axis=1)


_KVPG = 2


def _rope_wide(x, cosw, sinw, width):
    lanes = lax.broadcasted_iota(jnp.int32, (T, width), 1)
    xs = jnp.where((lanes % HD) < HALF,
                   pltpu.roll(x, width - HALF, 1), pltpu.roll(x, HALF, 1))
    return x * cosw + xs * sinw


def _attn_body(q_ref, k_ref, v_ref, cosq_ref, sinq_ref, cosk_ref, sink_ref,
               sinks_ref, o_ref):
    qw = _KVPG * GS * HD
    kvw = _KVPG * HD
    q_rot = _rope_wide(q_ref[...], cosq_ref[...], sinq_ref[...], qw).astype(BF)
    k_rot = _rope_wide(k_ref[...], cosk_ref[...], sink_ref[...], kvw).astype(BF)
    v = v_ref[...].astype(BF)
    ii = lax.broadcasted_iota(jnp.int32, (T, T), 0)
    jj = lax.broadcasted_iota(jnp.int32, (T, T), 1)
    mask = (jj <= ii) & ((ii - jj) <= SW)
    for kk in range(_KVPG):
        kh = k_rot[:, kk * HD:(kk + 1) * HD]
        vh = v[:, kk * HD:(kk + 1) * HD]
        for i in range(GS):
            h = kk * GS + i
            qh = q_rot[:, h * HD:(h + 1) * HD]
            logits = lax.dot_general(qh, kh, (((1,), (1,)), ((), ())),
                                     preferred_element_type=F32) * SCALE
            logits = jnp.where(mask, logits, -1e30)
            s = sinks_ref[h, 0]
            m = jnp.maximum(jnp.max(logits, axis=1, keepdims=True), s)
            p = jnp.exp(logits - m)
            denom = jnp.sum(p, axis=1, keepdims=True) + jnp.exp(s - m)
            probs = (p / denom).astype(BF)
            o_ref[:, h * HD:(h + 1) * HD] = jnp.dot(
                probs, vh, preferred_element_type=F32)


def _attn_call(qkv, cos, sin, sinks):
    grid = (NKV // _KVPG,)
    qw = _KVPG * GS * HD
    kvw = _KVPG * HD
    kv_col0 = (NH * HD) // kvw
    v_col0 = (NH * HD + NKV * HD) // kvw
    cos2 = jnp.concatenate([cos, cos], axis=1)
    sin2 = jnp.concatenate([-sin, sin], axis=1)
    cosq = jnp.tile(cos2, (1, qw // HD))
    sinq = jnp.tile(sin2, (1, qw // HD))
    cosk = jnp.tile(cos2, (1, kvw // HD))
    sink = jnp.tile(sin2, (1, kvw // HD))
    return pl.pallas_call(
        _attn_body,
        grid=grid,
        in_specs=[
            pl.BlockSpec((T, qw), lambda g: (0, g)),
            pl.BlockSpec((T, kvw), lambda g: (0, kv_col0 + g)),
            pl.BlockSpec((T, kvw), lambda g: (0, v_col0 + g)),
            pl.BlockSpec((T, qw), lambda g: (0, 0)),
            pl.BlockSpec((T, qw), lambda g: (0, 0)),
            pl.BlockSpec((T, kvw), lambda g: (0, 0)),
            pl.BlockSpec((T, kvw), lambda g: (0, 0)),
            pl.BlockSpec((_KVPG * GS, 1), lambda g: (g, 0)),
        ],
        out_specs=pl.BlockSpec((T, qw), lambda g: (0, g)),
        out_shape=jax.ShapeDtypeStruct((T, NH * HD), F32),
    )(qkv, qkv, qkv, cosq, sinq, cosk, sink, sinks.reshape(NH, 1))


def _oproj_body(attn_ref, w_ref, b_ref, hid_ref, o_ref):
    a = attn_ref[...].astype(BF)
    w = w_ref[...].astype(BF)
    acc = lax.dot_general(a, w, (((1,), (1,)), ((), ())),
                          preferred_element_type=F32)
    o_ref[...] = hid_ref[...] + acc + b_ref[...]


def _oproj_call(attn, o_w, o_b, hidden):
    grid = (D // OW_BLK,)
    return pl.pallas_call(
        _oproj_body,
        grid=grid,
        in_specs=[
            pl.BlockSpec((T, NH * HD), lambda b: (0, 0)),
            pl.BlockSpec((OW_BLK, NH * HD), lambda b: (b, 0)),
            pl.BlockSpec((1, OW_BLK), lambda b: (0, b)),
            pl.BlockSpec((T, OW_BLK), lambda b: (0, b)),
        ],
        out_specs=pl.BlockSpec((T, OW_BLK), lambda b: (0, b)),
        out_shape=jax.ShapeDtypeStruct((T, D), F32),
    )(attn, o_w, o_b.reshape(1, D), hidden)


def _router_body(res_ref, ln_ref, rw_ref, rb_ref, h2_ref, rlt_ref):
    x = res_ref[...]
    h2 = x * _rms_scale(x) * ln_ref[...]
    h2_ref[...] = h2
    rw = rw_ref[...].astype(BF)
    rlt_ref[...] = lax.dot_general(rw, h2.astype(BF), (((1,), (1,)), ((), ())),
                                   preferred_element_type=F32) + rb_ref[...]


def _router_call(res1, ln2_w, router_w, router_b):
    return pl.pallas_call(
        _router_body,
        out_shape=[
            jax.ShapeDtypeStruct((T, D), F32),
            jax.ShapeDtypeStruct((E, T), F32),
        ],
    )(res1, ln2_w.reshape(1, D), router_w, router_b.reshape(E, 1))


_NC = 2
_NTW = 16
_TKW = T // _NTW


def _route_body(rl_hbm, c_hbm, rl_v, c_v):
    wid = lax.axis_index("s") * _NC + lax.axis_index("c")

    @pl.when(wid < _NTW)
    def _work():
        base = wid * _TKW
        pltpu.sync_copy(rl_hbm, rl_v)
        neg = jnp.full((_TKW,), -1e30, F32)
        m1 = neg
        m2 = neg
        i1 = jnp.zeros((_TKW,), jnp.int32)
        i2 = jnp.full((_TKW,), -1, jnp.int32)
        for e in range(E):
            v = rl_v[pl.ds(e * T + base, _TKW)]
            cand_i = jnp.where(v > m1, i1, e)
            cand_v = jnp.minimum(v, m1)
            i1 = jnp.where(v > m1, e, i1)
            m1 = jnp.maximum(m1, v)
            i2 = jnp.where(cand_v > m2, cand_i, i2)
            m2 = jnp.maximum(m2, cand_v)
        ed = jnp.exp(m2 - m1)
        w1 = 1.0 / (1.0 + ed)
        w2 = 1.0 - w1
        zero = jnp.zeros((_TKW,), F32)
        for e in range(E):
            ce = jnp.where(i1 == e, w1, zero) + jnp.where(i2 == e, w2, zero)
            c_v[pl.ds(e * _TKW, _TKW)] = ce
        for e in range(E):
            pltpu.sync_copy(c_v.at[pl.ds(e * _TKW, _TKW)],
                            c_hbm.at[pl.ds(e * T + base, _TKW)])


def _route_call(rl_t_flat):
    mesh = plsc.VectorSubcoreMesh(core_axis_name="c", subcore_axis_name="s")
    fn = functools.partial(
        pl.kernel, mesh=mesh,
        out_type=jax.ShapeDtypeStruct((E * T,), F32),
        scratch_types=[
            pltpu.VMEM((E * T,), F32),
            pltpu.VMEM((E * _TKW,), F32),
        ],
    )(_route_body)
    return fn(rl_t_flat)


def _moe_body(h2_ref, res_ref, c_ref, wg_ref, wu_ref, bg_ref, bu_ref,
              wd_ref, bd_ref, o_ref):
    e = pl.program_id(0)
    f = pl.program_id(1)

    @pl.when((e == 0) & (f == 0))
    def _init():
        o_ref[...] = res_ref[...]

    h2 = h2_ref[...].astype(BF)
    gate = lax.dot_general(h2, wg_ref[0].astype(BF), (((1,), (0,)), ((), ())),
                           preferred_element_type=F32) + bg_ref[0]
    up = lax.dot_general(h2, wu_ref[0].astype(BF), (((1,), (0,)), ((), ())),
                         preferred_element_type=F32) + bu_ref[0]
    gate = jnp.minimum(gate, LIMIT)
    up = jnp.clip(up, -LIMIT, LIMIT)
    act = ((up + 1.0) * (gate * jax.nn.sigmoid(ALPHA * gate))).astype(BF)
    downp = lax.dot_general(act, wd_ref[0].astype(BF), (((1,), (0,)), ((), ())),
                            preferred_element_type=F32)

    eidx = lax.broadcasted_iota(jnp.int32, (T, E), 1)
    ccol = jnp.sum(jnp.where(eidx == e, c_ref[...], 0.0),
                   axis=1, keepdims=True)
    contrib = ccol * downp

    @pl.when(f == 0)
    def _bias():
        o_ref[...] += ccol * bd_ref[0]

    o_ref[...] += contrib


def _moe_call(h2, res1, c, w_gate_up, b_gate_up, w_down, b_down):
    grid = (E, NF)
    return pl.pallas_call(
        _moe_body,
        grid=grid,
        in_specs=[
            pl.BlockSpec((T, D), lambda e, f: (0, 0)),
            pl.BlockSpec((T, D), lambda e, f: (0, 0)),
            pl.BlockSpec((T, E), lambda e, f: (0, 0)),
            pl.BlockSpec((1, D, F_BLK), lambda e, f: (e, 0, f)),
            pl.BlockSpec((1, D, F_BLK), lambda e, f: (e, 0, NF + f)),
            pl.BlockSpec((1, 1, F_BLK), lambda e, f: (e, 0, f)),
            pl.BlockSpec((1, 1, F_BLK), lambda e, f: (e, 0, NF + f)),
            pl.BlockSpec((1, F_BLK, D), lambda e, f: (e, f, 0)),
            pl.BlockSpec((1, 1, D), lambda e, f: (e, 0, 0)),
        ],
        out_specs=pl.BlockSpec((T, D), lambda e, f: (0, 0)),
        out_shape=jax.ShapeDtypeStruct((T, D), F32),
    )(h2, res1, c, w_gate_up, w_gate_up,
      b_gate_up.reshape(E, 1, 2 * INTER), b_gate_up.reshape(E, 1, 2 * INTER),
      w_down, b_down.reshape(E, 1, D))


def kernel(positions, hidden_states, ln1_w, qkv_w, qkv_b, sinks, o_w, o_b,
           ln2_w, router_w, router_b, w_gate_up, b_gate_up, w_down, b_down):
    inv = 1.0 / (THETA ** (jnp.arange(HALF, dtype=F32) / HALF))
    ang = positions.astype(F32)[:, None] * inv[None, :]
    cos = jnp.cos(ang)
    sin = jnp.sin(ang)

    qkv = _qkv_call(hidden_states, ln1_w, qkv_w, qkv_b)
    attn = _attn_call(qkv, cos, sin, sinks)
    res1 = _oproj_call(attn, o_w, o_b, hidden_states)
    h2, rl_t = _router_call(res1, ln2_w, router_w, router_b)
    c = _route_call(rl_t.reshape(-1)).reshape(E, T).T
    return _moe_call(h2, res1, c, w_gate_up, b_gate_up, w_down, b_down)
